# baseline (speedup 1.0000x reference)
import numpy as np
import jax
import jax.numpy as jnp
from jax.experimental import pallas as pl
from jax.experimental.pallas import tpu as pltpu

DIM = 4096
E = 64
K = 8
IMP_COEFF = 0.01
LOAD_COEFF = 0.01
EPS = 1e-9
TB = 512
RC = 64

_INV_SQRT2 = np.float32(1.0 / np.sqrt(2.0))


def _cv_sq(v):
    m = jnp.mean(v)
    var = jnp.mean((v - m) ** 2)
    return var / (m * m + np.float32(EPS))


def _router_body(x_ref, w_ref, topi_ref, wts_ref, prio_ref, aux_ref,
                 imp_ref, load_ref, acc_ref):
    i = pl.program_id(0)
    nb = pl.num_programs(0)

    iota = jax.lax.broadcasted_iota(jnp.int32, (RC, E), 1)
    imp_part = jnp.zeros((1, E), jnp.float32)
    load_part = jnp.zeros((1, E), jnp.float32)

    for c in range(TB // RC):
        r0 = c * RC
        logits = acc_ref[r0:r0 + RC, :E]
        nlog = acc_ref[r0:r0 + RC, E:]
        noise_std = jax.nn.softplus(nlog) + np.float32(EPS)

        m = jnp.max(logits, axis=-1, keepdims=True)
        p = jnp.exp(logits - m)
        s = jnp.sum(p, axis=-1, keepdims=True)
        inv_s = 1.0 / s
        prio_ref[r0:r0 + RC, :] = inv_s
        imp_part = imp_part + jnp.sum(p * inv_s, axis=0, keepdims=True)

        work = logits
        sel = jnp.zeros((RC, E), jnp.bool_)
        topv = []
        topidx = []
        for j in range(K + 1):
            mj = jnp.max(work, axis=-1, keepdims=True)
            idx = jnp.argmax(work, axis=-1).astype(jnp.int32)
            oh = iota == idx[:, None]
            topv.append(mj)
            if j < K:
                topidx.append(idx[:, None])
                sel = jnp.logical_or(sel, oh)
            work = jnp.where(oh, -jnp.inf, work)

        tv = jnp.concatenate(topv, axis=1)
        topi_ref[r0:r0 + RC, :] = jnp.concatenate(topidx, axis=1)
        wts_ref[r0:r0 + RC, :] = jax.nn.softmax(tv[:, :K], axis=-1)

        v_k = tv[:, K - 1:K]
        v_kp = tv[:, K:K + 1]
        kth = jnp.where(sel, v_kp, v_k)
        z = (logits - kth) / noise_std
        phi = 0.5 * (1.0 + jax.lax.erf(z * _INV_SQRT2))
        load_part = load_part + jnp.sum(phi, axis=0, keepdims=True)

    @pl.when(i == 1)
    def _():
        imp_ref[...] = imp_part
        load_ref[...] = load_part

    @pl.when(i > 1)
    def _():
        imp_ref[...] += imp_part
        load_ref[...] += load_part

    @pl.when(i == nb - 1)
    def _():
        aux = (np.float32(IMP_COEFF) * _cv_sq(imp_ref[...])
               + np.float32(LOAD_COEFF) * _cv_sq(load_ref[...]))
        aux_ref[...] = jnp.full((1, 1), aux, jnp.float32)

    acc_ref[...] = jnp.dot(x_ref[...], w_ref[...],
                           preferred_element_type=jnp.float32)


def kernel(x, W_gate, W_noise):
    orig_shape = x.shape
    x2 = x.reshape(-1, orig_shape[-1])
    n = x2.shape[0]
    wcat = jnp.concatenate([W_gate.T, W_noise.T], axis=1)
    nb = n // TB
    last = nb - 1

    def prev(i):
        return jnp.where(i > 0, i - 1, 0)

    topi, wts, prio, aux, _, _ = pl.pallas_call(
        _router_body,
        grid=(nb + 1,),
        in_specs=[
            pl.BlockSpec((TB, DIM), lambda i: (jnp.minimum(i, last), 0)),
            pl.BlockSpec((DIM, 2 * E), lambda i: (0, 0)),
        ],
        out_specs=[
            pl.BlockSpec((TB, K), lambda i: (prev(i), 0)),
            pl.BlockSpec((TB, K), lambda i: (prev(i), 0)),
            pl.BlockSpec((TB, 1), lambda i: (prev(i), 0)),
            pl.BlockSpec((1, 1), lambda i: (0, 0)),
            pl.BlockSpec((1, E), lambda i: (0, 0)),
            pl.BlockSpec((1, E), lambda i: (0, 0)),
        ],
        out_shape=[
            jax.ShapeDtypeStruct((n, K), jnp.int32),
            jax.ShapeDtypeStruct((n, K), jnp.float32),
            jax.ShapeDtypeStruct((n, 1), jnp.float32),
            jax.ShapeDtypeStruct((1, 1), jnp.float32),
            jax.ShapeDtypeStruct((1, E), jnp.float32),
            jax.ShapeDtypeStruct((1, E), jnp.float32),
        ],
        scratch_shapes=[pltpu.VMEM((TB, 2 * E), jnp.float32)],
        compiler_params=pltpu.CompilerParams(
            dimension_semantics=("arbitrary",),
        ),
    )(x2, wcat)

    leading = orig_shape[:-1]
    return (topi.reshape(leading + (K,)),
            wts.reshape(leading + (K,)),
            prio.reshape(n),
            aux.reshape(()))


# speedup vs baseline: 1.0908x; 1.0908x over previous
import numpy as np
import jax
import jax.numpy as jnp
from jax.experimental import pallas as pl
from jax.experimental.pallas import tpu as pltpu

DIM = 4096
E = 64
K = 8
IMP_COEFF = 0.01
LOAD_COEFF = 0.01
EPS = 1e-9
TB = 512
RC = 64

_INV_SQRT2 = np.float32(1.0 / np.sqrt(2.0))


def _cv_sq(v):
    m = jnp.mean(v)
    var = jnp.mean((v - m) ** 2)
    return var / (m * m + np.float32(EPS))


def _router_body(x_ref, w_ref, topi_ref, wts_ref, prio_ref, aux_ref,
                 imp_ref, load_ref, acc_ref):
    i = pl.program_id(0)
    nb = pl.num_programs(0)

    iota = jax.lax.broadcasted_iota(jnp.int32, (RC, E), 1)
    imp_part = jnp.zeros((1, E), jnp.float32)
    load_part = jnp.zeros((1, E), jnp.float32)

    for c in range(TB // RC):
        r0 = c * RC

        logits = acc_ref[r0:r0 + RC, :E]
        m = jnp.max(logits, axis=-1, keepdims=True)
        p = jnp.exp(logits - m)
        s = jnp.sum(p, axis=-1, keepdims=True)
        inv_s = 1.0 / s
        prio_ref[r0:r0 + RC, :] = inv_s
        imp_part = imp_part + jnp.sum(p * inv_s, axis=0, keepdims=True)

        sel = jnp.zeros((RC, E), jnp.bool_)
        topv = []
        topidx = []
        for j in range(K + 1):
            work = jnp.where(sel, -jnp.inf, acc_ref[r0:r0 + RC, :E])
            mj = jnp.max(work, axis=-1, keepdims=True)
            cand = jnp.where(work == mj, iota, E)
            idx = jnp.min(cand, axis=-1, keepdims=True)
            topv.append(mj)
            if j < K:
                topidx.append(idx)
                sel = jnp.logical_or(sel, iota == idx)

        tv = jnp.concatenate(topv, axis=1)
        topi_ref[r0:r0 + RC, :] = jnp.concatenate(topidx, axis=1)
        wts_ref[r0:r0 + RC, :] = jax.nn.softmax(tv[:, :K], axis=-1)

        v_k = tv[:, K - 1:K]
        v_kp = tv[:, K:K + 1]
        kth = jnp.where(sel, v_kp, v_k)
        noise_std = jax.nn.softplus(acc_ref[r0:r0 + RC, E:]) + np.float32(EPS)
        z = (acc_ref[r0:r0 + RC, :E] - kth) / noise_std
        phi = 0.5 * (1.0 + jax.lax.erf(z * _INV_SQRT2))
        load_part = load_part + jnp.sum(phi, axis=0, keepdims=True)

    @pl.when(i == 1)
    def _():
        imp_ref[...] = imp_part
        load_ref[...] = load_part

    @pl.when(i > 1)
    def _():
        imp_ref[...] += imp_part
        load_ref[...] += load_part

    @pl.when(i == nb - 1)
    def _():
        aux = (np.float32(IMP_COEFF) * _cv_sq(imp_ref[...])
               + np.float32(LOAD_COEFF) * _cv_sq(load_ref[...]))
        aux_ref[...] = jnp.full((1, 1), aux, jnp.float32)

    acc_ref[...] = jnp.dot(x_ref[...], w_ref[...],
                           preferred_element_type=jnp.float32)


def kernel(x, W_gate, W_noise):
    orig_shape = x.shape
    x2 = x.reshape(-1, orig_shape[-1])
    n = x2.shape[0]
    wcat = jnp.concatenate([W_gate.T, W_noise.T], axis=1)
    nb = n // TB
    last = nb - 1

    def prev(i):
        return jnp.where(i > 0, i - 1, 0)

    topi, wts, prio, aux, _, _ = pl.pallas_call(
        _router_body,
        grid=(nb + 1,),
        in_specs=[
            pl.BlockSpec((TB, DIM), lambda i: (jnp.minimum(i, last), 0)),
            pl.BlockSpec((DIM, 2 * E), lambda i: (0, 0)),
        ],
        out_specs=[
            pl.BlockSpec((TB, K), lambda i: (prev(i), 0)),
            pl.BlockSpec((TB, K), lambda i: (prev(i), 0)),
            pl.BlockSpec((TB, 1), lambda i: (prev(i), 0)),
            pl.BlockSpec((1, 1), lambda i: (0, 0)),
            pl.BlockSpec((1, E), lambda i: (0, 0)),
            pl.BlockSpec((1, E), lambda i: (0, 0)),
        ],
        out_shape=[
            jax.ShapeDtypeStruct((n, K), jnp.int32),
            jax.ShapeDtypeStruct((n, K), jnp.float32),
            jax.ShapeDtypeStruct((n, 1), jnp.float32),
            jax.ShapeDtypeStruct((1, 1), jnp.float32),
            jax.ShapeDtypeStruct((1, E), jnp.float32),
            jax.ShapeDtypeStruct((1, E), jnp.float32),
        ],
        scratch_shapes=[pltpu.VMEM((TB, 2 * E), jnp.float32)],
        compiler_params=pltpu.CompilerParams(
            dimension_semantics=("arbitrary",),
        ),
    )(x2, wcat)

    leading = orig_shape[:-1]
    return (topi.reshape(leading + (K,)),
            wts.reshape(leading + (K,)),
            prio.reshape(n),
            aux.reshape(()))


# speedup vs baseline: 2.4455x; 2.2419x over previous
import numpy as np
import jax
import jax.numpy as jnp
from jax.experimental import pallas as pl
from jax.experimental.pallas import tpu as pltpu

DIM = 4096
E = 64
K = 8
IMP_COEFF = 0.01
LOAD_COEFF = 0.01
EPS = 1e-9
TB = 512
TC = 128

_INV_SQRT2 = np.float32(1.0 / np.sqrt(2.0))


def _cv_sq(v):
    m = jnp.mean(v)
    var = jnp.mean((v - m) ** 2)
    return var / (m * m + np.float32(EPS))


def _router_body(x_ref, w_ref, topi_ref, wts_ref, prio_ref, aux_ref,
                 acc_ref, imp_ref, load_ref):
    i = pl.program_id(0)
    nb = pl.num_programs(0)

    riota = jax.lax.broadcasted_iota(jnp.int32, (E, TC), 0)
    imp_part = jnp.zeros((E, TC), jnp.float32)
    load_part = jnp.zeros((E, TC), jnp.float32)

    for c in range(TB // TC):
        t0 = c * TC

        logits = acc_ref[:E, t0:t0 + TC]
        m = jnp.max(logits, axis=0, keepdims=True)
        p = jnp.exp(logits - m)
        inv_s = 1.0 / jnp.sum(p, axis=0, keepdims=True)
        prio_ref[:, t0:t0 + TC] = inv_s
        imp_part = imp_part + p * inv_s

        sel = jnp.zeros((E, TC), jnp.bool_)
        topv = []
        topidx = []
        for j in range(K + 1):
            work = jnp.where(sel, -jnp.inf, acc_ref[:E, t0:t0 + TC])
            mj = jnp.max(work, axis=0, keepdims=True)
            cand = jnp.where(work == mj, riota, E)
            idx = jnp.min(cand, axis=0, keepdims=True)
            topv.append(mj)
            if j < K:
                topidx.append(idx)
                sel = jnp.logical_or(sel, riota == idx)

        tv = jnp.concatenate(topv, axis=0)
        topi_ref[:, t0:t0 + TC] = jnp.concatenate(topidx, axis=0)
        wts_ref[:, t0:t0 + TC] = jax.nn.softmax(tv[:K], axis=0)

        v_k = tv[K - 1:K]
        v_kp = tv[K:K + 1]
        kth = jnp.where(sel, v_kp, v_k)
        nstd = jax.nn.softplus(acc_ref[E:, t0:t0 + TC]) + np.float32(EPS)
        z = (acc_ref[:E, t0:t0 + TC] - kth) / nstd
        phi = 0.5 * (1.0 + jax.lax.erf(z * _INV_SQRT2))
        load_part = load_part + phi

    @pl.when(i == 1)
    def _():
        imp_ref[...] = imp_part
        load_ref[...] = load_part

    @pl.when(i > 1)
    def _():
        imp_ref[...] += imp_part
        load_ref[...] += load_part

    @pl.when(i == nb - 1)
    def _():
        imp = jnp.sum(imp_ref[...], axis=1)
        load = jnp.sum(load_ref[...], axis=1)
        aux = (np.float32(IMP_COEFF) * _cv_sq(imp)
               + np.float32(LOAD_COEFF) * _cv_sq(load))
        aux_ref[...] = jnp.full((1, 1), aux, jnp.float32)

    acc_ref[...] = jax.lax.dot_general(
        w_ref[...], x_ref[...],
        dimension_numbers=(((1,), (1,)), ((), ())),
        preferred_element_type=jnp.float32)


def kernel(x, W_gate, W_noise):
    orig_shape = x.shape
    x2 = x.reshape(-1, orig_shape[-1])
    n = x2.shape[0]
    wcat = jnp.concatenate([W_gate, W_noise], axis=0)
    nb = n // TB
    last = nb - 1

    def prev(i):
        return jnp.where(i > 0, i - 1, 0)

    topi_t, wts_t, prio_t, aux = pl.pallas_call(
        _router_body,
        grid=(nb + 1,),
        in_specs=[
            pl.BlockSpec((TB, DIM), lambda i: (jnp.minimum(i, last), 0)),
            pl.BlockSpec((2 * E, DIM), lambda i: (0, 0)),
        ],
        out_specs=[
            pl.BlockSpec((K, TB), lambda i: (0, prev(i))),
            pl.BlockSpec((K, TB), lambda i: (0, prev(i))),
            pl.BlockSpec((1, TB), lambda i: (0, prev(i))),
            pl.BlockSpec((1, 1), lambda i: (0, 0)),
        ],
        out_shape=[
            jax.ShapeDtypeStruct((K, n), jnp.int32),
            jax.ShapeDtypeStruct((K, n), jnp.float32),
            jax.ShapeDtypeStruct((1, n), jnp.float32),
            jax.ShapeDtypeStruct((1, 1), jnp.float32),
        ],
        scratch_shapes=[
            pltpu.VMEM((2 * E, TB), jnp.float32),
            pltpu.VMEM((E, TC), jnp.float32),
            pltpu.VMEM((E, TC), jnp.float32),
        ],
        compiler_params=pltpu.CompilerParams(
            dimension_semantics=("arbitrary",),
        ),
    )(x2, wcat)

    leading = orig_shape[:-1]
    return (topi_t.T.reshape(leading + (K,)),
            wts_t.T.reshape(leading + (K,)),
            prio_t.reshape(n),
            aux.reshape(()))


# speedup vs baseline: 2.6071x; 1.0661x over previous
import numpy as np
import jax
import jax.numpy as jnp
from jax.experimental import pallas as pl
from jax.experimental.pallas import tpu as pltpu

DIM = 4096
E = 64
K = 8
IMP_COEFF = 0.01
LOAD_COEFF = 0.01
EPS = 1e-9
TB = 1024
TC = 128

_INV_SQRT2 = np.float32(1.0 / np.sqrt(2.0))


def _cv_sq(v):
    m = jnp.mean(v)
    var = jnp.mean((v - m) ** 2)
    return var / (m * m + np.float32(EPS))


def _router_body(x_ref, w_ref, topi_ref, wts_ref, prio_ref, aux_ref,
                 acc_ref, imp_ref, load_ref):
    i = pl.program_id(0)
    nb = pl.num_programs(0)

    riota = jax.lax.broadcasted_iota(jnp.int32, (E, TC), 0)
    imp_part = jnp.zeros((E, TC), jnp.float32)
    load_part = jnp.zeros((E, TC), jnp.float32)

    for c in range(TB // TC):
        t0 = c * TC

        logits = acc_ref[:E, t0:t0 + TC]
        m = jnp.max(logits, axis=0, keepdims=True)
        p = jnp.exp(logits - m)
        inv_s = 1.0 / jnp.sum(p, axis=0, keepdims=True)
        prio_ref[:, t0:t0 + TC] = inv_s
        imp_part = imp_part + p * inv_s

        sel = jnp.zeros((E, TC), jnp.bool_)
        topv = []
        topidx = []
        for j in range(K + 1):
            work = jnp.where(sel, -jnp.inf, acc_ref[:E, t0:t0 + TC])
            mj = jnp.max(work, axis=0, keepdims=True)
            cand = jnp.where(work == mj, riota, E)
            idx = jnp.min(cand, axis=0, keepdims=True)
            topv.append(mj)
            if j < K:
                topidx.append(idx)
                sel = jnp.logical_or(sel, riota == idx)

        tv = jnp.concatenate(topv, axis=0)
        topi_ref[:, t0:t0 + TC] = jnp.concatenate(topidx, axis=0)
        wts_ref[:, t0:t0 + TC] = jax.nn.softmax(tv[:K], axis=0)

        v_k = tv[K - 1:K]
        v_kp = tv[K:K + 1]
        kth = jnp.where(sel, v_kp, v_k)
        nstd = jax.nn.softplus(acc_ref[E:, t0:t0 + TC]) + np.float32(EPS)
        z = (acc_ref[:E, t0:t0 + TC] - kth) / nstd
        phi = 0.5 * (1.0 + jax.lax.erf(z * _INV_SQRT2))
        load_part = load_part + phi

    @pl.when(i == 1)
    def _():
        imp_ref[...] = imp_part
        load_ref[...] = load_part

    @pl.when(i > 1)
    def _():
        imp_ref[...] += imp_part
        load_ref[...] += load_part

    @pl.when(i == nb - 1)
    def _():
        imp = jnp.sum(imp_ref[...], axis=1)
        load = jnp.sum(load_ref[...], axis=1)
        aux = (np.float32(IMP_COEFF) * _cv_sq(imp)
               + np.float32(LOAD_COEFF) * _cv_sq(load))
        aux_ref[...] = jnp.full((1, 1), aux, jnp.float32)

    acc_ref[...] = jax.lax.dot_general(
        w_ref[...], x_ref[...],
        dimension_numbers=(((1,), (1,)), ((), ())),
        preferred_element_type=jnp.float32)


def kernel(x, W_gate, W_noise):
    orig_shape = x.shape
    x2 = x.reshape(-1, orig_shape[-1])
    n = x2.shape[0]
    wcat = jnp.concatenate([W_gate, W_noise], axis=0)
    nb = n // TB
    last = nb - 1

    def prev(i):
        return jnp.where(i > 0, i - 1, 0)

    topi_t, wts_t, prio_t, aux = pl.pallas_call(
        _router_body,
        grid=(nb + 1,),
        in_specs=[
            pl.BlockSpec((TB, DIM), lambda i: (jnp.minimum(i, last), 0)),
            pl.BlockSpec((2 * E, DIM), lambda i: (0, 0)),
        ],
        out_specs=[
            pl.BlockSpec((K, TB), lambda i: (0, prev(i))),
            pl.BlockSpec((K, TB), lambda i: (0, prev(i))),
            pl.BlockSpec((1, TB), lambda i: (0, prev(i))),
            pl.BlockSpec((1, 1), lambda i: (0, 0)),
        ],
        out_shape=[
            jax.ShapeDtypeStruct((K, n), jnp.int32),
            jax.ShapeDtypeStruct((K, n), jnp.float32),
            jax.ShapeDtypeStruct((1, n), jnp.float32),
            jax.ShapeDtypeStruct((1, 1), jnp.float32),
        ],
        scratch_shapes=[
            pltpu.VMEM((2 * E, TB), jnp.float32),
            pltpu.VMEM((E, TC), jnp.float32),
            pltpu.VMEM((E, TC), jnp.float32),
        ],
        compiler_params=pltpu.CompilerParams(
            dimension_semantics=("arbitrary",),
        ),
    )(x2, wcat)

    leading = orig_shape[:-1]
    return (topi_t.T.reshape(leading + (K,)),
            wts_t.T.reshape(leading + (K,)),
            prio_t.reshape(n),
            aux.reshape(()))
